# baseline (speedup 1.0000x reference)
import jax
import jax.numpy as jnp
from jax import lax
from jax.experimental import pallas as pl
from jax.experimental.pallas import tpu as pltpu
from jax.experimental.pallas import tpu_sc as plsc
import functools

TOTAL = 32768
B = 16
D = 128
LANES = 16
NC = 2
NS = 16
NW = NC * NS

SC_ROWS = 4096
ROWS_PER_W = SC_ROWS // NW
CHUNK = 64
NCHUNK = ROWS_PER_W // CHUNK
DV = D // LANES

TC_BLK = 4096
TC_ROWS = TOTAL - SC_ROWS


def _sc_partial_sums(h_hbm, ptr_hbm, out_hbm, ptr_v, buf0, buf1, acc,
                     shared, ptr_s, sem0, sem1):
  cid = lax.axis_index("c")
  sid = lax.axis_index("s")
  wid = sid * NC + cid
  base = wid * ROWS_PER_W
  bufs = (buf0, buf1)
  sems = (sem0, sem1)

  pending = {}
  for c in range(min(2, NCHUNK)):
    pending[c] = pltpu.async_copy(
        h_hbm.at[pl.ds(base + c * CHUNK, CHUNK)], bufs[c % 2], sems[c % 2])

  pltpu.sync_copy(ptr_hbm.at[pl.ds(0, LANES)], ptr_v)
  ptr_vec = ptr_v[...]
  for b in range(B):
    ptr_s[b] = ptr_vec[b]
  ptr_s[B] = jnp.int32(TOTAL)

  zero = jnp.zeros((LANES,), jnp.float32)
  for b in range(B):
    for j in range(DV):
      acc[b, pl.ds(j * LANES, LANES)] = zero

  @pl.when(sid == 0)
  def _():
    pltpu.sync_copy(acc, shared)

  plsc.subcore_barrier()

  for c in range(NCHUNK):
    clo = base + c * CHUNK
    buf = bufs[c % 2]
    pending[c].wait()

    def bag_body(b, _):
      s_loc = jnp.clip(ptr_s[b] - clo, 0, CHUNK)
      e_loc = jnp.clip(ptr_s[b + 1] - clo, 0, CHUNK)

      @pl.when(e_loc > s_loc)
      def _():
        @plsc.parallel_loop(s_loc, e_loc, step=1, unroll=2,
                            carry=(zero,) * DV)
        def sums(r, carry):
          return tuple(carry[j] + buf[r, pl.ds(j * LANES, LANES)]
                       for j in range(DV))
        for j in range(DV):
          acc[b, pl.ds(j * LANES, LANES)] = (
              acc[b, pl.ds(j * LANES, LANES)] + sums[j])
      return 0

    lax.fori_loop(0, B, bag_body, 0)

    if c + 2 < NCHUNK:
      pending[c + 2] = pltpu.async_copy(
          h_hbm.at[pl.ds(base + (c + 2) * CHUNK, CHUNK)], buf, sems[c % 2])

  pltpu.sync_copy(acc, shared.at[lax.iota(jnp.int32, B)], add=True)
  plsc.subcore_barrier()

  @pl.when(sid == 0)
  def _():
    pltpu.sync_copy(shared, out_hbm.at[cid])


@functools.partial(
    pl.kernel,
    out_type=jax.ShapeDtypeStruct((NC, B, D), jnp.float32),
    mesh=plsc.VectorSubcoreMesh(core_axis_name="c", subcore_axis_name="s"),
    scratch_types=[
        pltpu.VMEM((LANES,), jnp.int32),
        pltpu.VMEM((CHUNK, D), jnp.float32),
        pltpu.VMEM((CHUNK, D), jnp.float32),
        pltpu.VMEM((B, D), jnp.float32),
        pltpu.VMEM_SHARED((B, D), jnp.float32),
        pltpu.SMEM((B + 1,), jnp.int32),
        pltpu.SemaphoreType.DMA,
        pltpu.SemaphoreType.DMA,
    ],
)
def _partial_sums(h_hbm, ptr_hbm, out_hbm, ptr_v, buf0, buf1, acc,
                  shared, ptr_s, sem0, sem1):
  _sc_partial_sums(h_hbm, ptr_hbm, out_hbm, ptr_v, buf0, buf1, acc,
                   shared, ptr_s, sem0, sem1)


def _tc_rowsum_body(ptr_ref, h_ref, out_ref):
  i = pl.program_id(0)
  rows = (SC_ROWS + i * TC_BLK
          + jax.lax.broadcasted_iota(jnp.int32, (1, TC_BLK), 1))
  lower = jnp.stack([ptr_ref[b] for b in range(B)])[:, None]
  upper = jnp.stack([ptr_ref[b + 1] for b in range(B)])[:, None]
  mask = ((rows >= lower) & (rows < upper)).astype(jnp.float32)
  partial = jax.lax.dot_general(
      mask, h_ref[...], (((1,), (0,)), ((), ())),
      preferred_element_type=jnp.float32)

  @pl.when(i == 0)
  def _():
    out_ref[...] = jnp.zeros_like(out_ref)

  out_ref[...] += partial


def _tc_rowsum(H, bag_ptr):
  return pl.pallas_call(
      _tc_rowsum_body,
      grid=(TC_ROWS // TC_BLK,),
      in_specs=[
          pl.BlockSpec(memory_space=pltpu.SMEM),
          pl.BlockSpec((TC_BLK, D), lambda i: (i + SC_ROWS // TC_BLK, 0)),
      ],
      out_specs=pl.BlockSpec((B, D), lambda i: (0, 0)),
      out_shape=jax.ShapeDtypeStruct((B, D), jnp.float32),
  )(bag_ptr, H)


def _combine_body(sc_ref, tc_ref, ptr_ref, out_ref):
  sums = jnp.sum(sc_ref[...], axis=0) + tc_ref[...]
  cnt = jnp.stack([ptr_ref[b + 1] - ptr_ref[b] for b in range(B)])
  denom = jnp.maximum(cnt.astype(jnp.float32), 1.0)[:, None]
  out_ref[...] = sums / denom


def kernel(H, bag_ptr):
  sc_partial = _partial_sums(H, bag_ptr)
  tc_partial = _tc_rowsum(H, bag_ptr)
  out = pl.pallas_call(
      _combine_body,
      out_shape=jax.ShapeDtypeStruct((B, D), jnp.float32),
      in_specs=[
          pl.BlockSpec(memory_space=pltpu.VMEM),
          pl.BlockSpec(memory_space=pltpu.VMEM),
          pl.BlockSpec(memory_space=pltpu.SMEM),
      ],
      out_specs=pl.BlockSpec(memory_space=pltpu.VMEM),
  )(sc_partial, tc_partial, bag_ptr)
  return out


# speedup vs baseline: 1.0304x; 1.0304x over previous
import jax
import jax.numpy as jnp
from jax import lax
from jax.experimental import pallas as pl
from jax.experimental.pallas import tpu as pltpu
from jax.experimental.pallas import tpu_sc as plsc
import functools

TOTAL = 32768
B = 16
D = 128
LANES = 16
NC = 1
NS = 16
NW = NC * NS

SC_ROWS = 4096
ROWS_PER_W = SC_ROWS // NW
CHUNK = 64
NCHUNK = ROWS_PER_W // CHUNK
DV = D // LANES

TC_BLK = 4096
TC_ROWS = TOTAL - SC_ROWS


def _sc_partial_sums(h_hbm, ptr_hbm, out_hbm, ptr_v, buf0, buf1, acc,
                     shared, ptr_s, sem0, sem1):
  cid = lax.axis_index("c")
  sid = lax.axis_index("s")
  wid = sid * NC + cid
  base = wid * ROWS_PER_W
  bufs = (buf0, buf1)
  sems = (sem0, sem1)

  pending = {}
  for c in range(min(2, NCHUNK)):
    pending[c] = pltpu.async_copy(
        h_hbm.at[pl.ds(base + c * CHUNK, CHUNK)], bufs[c % 2], sems[c % 2])

  pltpu.sync_copy(ptr_hbm.at[pl.ds(0, LANES)], ptr_v)
  ptr_vec = ptr_v[...]
  for b in range(B):
    ptr_s[b] = ptr_vec[b]
  ptr_s[B] = jnp.int32(TOTAL)

  zero = jnp.zeros((LANES,), jnp.float32)
  for b in range(B):
    for j in range(DV):
      acc[b, pl.ds(j * LANES, LANES)] = zero

  @pl.when(sid == 0)
  def _():
    pltpu.sync_copy(acc, shared)

  plsc.subcore_barrier()

  for c in range(NCHUNK):
    clo = base + c * CHUNK
    buf = bufs[c % 2]
    pending[c].wait()

    def bag_body(b, _):
      s_loc = jnp.clip(ptr_s[b] - clo, 0, CHUNK)
      e_loc = jnp.clip(ptr_s[b + 1] - clo, 0, CHUNK)

      @pl.when(e_loc > s_loc)
      def _():
        @plsc.parallel_loop(s_loc, e_loc, step=1, unroll=2,
                            carry=(zero,) * DV)
        def sums(r, carry):
          return tuple(carry[j] + buf[r, pl.ds(j * LANES, LANES)]
                       for j in range(DV))
        for j in range(DV):
          acc[b, pl.ds(j * LANES, LANES)] = (
              acc[b, pl.ds(j * LANES, LANES)] + sums[j])
      return 0

    lax.fori_loop(0, B, bag_body, 0)

    if c + 2 < NCHUNK:
      pending[c + 2] = pltpu.async_copy(
          h_hbm.at[pl.ds(base + (c + 2) * CHUNK, CHUNK)], buf, sems[c % 2])

  pltpu.sync_copy(acc, shared.at[lax.iota(jnp.int32, B)], add=True)
  plsc.subcore_barrier()

  @pl.when(sid == 0)
  def _():
    pltpu.sync_copy(shared, out_hbm.at[cid])


@functools.partial(
    pl.kernel,
    out_type=jax.ShapeDtypeStruct((NC, B, D), jnp.float32),
    mesh=plsc.VectorSubcoreMesh(core_axis_name="c", subcore_axis_name="s", num_cores=1),
    scratch_types=[
        pltpu.VMEM((LANES,), jnp.int32),
        pltpu.VMEM((CHUNK, D), jnp.float32),
        pltpu.VMEM((CHUNK, D), jnp.float32),
        pltpu.VMEM((B, D), jnp.float32),
        pltpu.VMEM_SHARED((B, D), jnp.float32),
        pltpu.SMEM((B + 1,), jnp.int32),
        pltpu.SemaphoreType.DMA,
        pltpu.SemaphoreType.DMA,
    ],
)
def _partial_sums(h_hbm, ptr_hbm, out_hbm, ptr_v, buf0, buf1, acc,
                  shared, ptr_s, sem0, sem1):
  _sc_partial_sums(h_hbm, ptr_hbm, out_hbm, ptr_v, buf0, buf1, acc,
                   shared, ptr_s, sem0, sem1)


def _tc_rowsum_body(ptr_ref, h_ref, out_ref):
  i = pl.program_id(0)
  rows = (SC_ROWS + i * TC_BLK
          + jax.lax.broadcasted_iota(jnp.int32, (1, TC_BLK), 1))
  lower = jnp.stack([ptr_ref[b] for b in range(B)])[:, None]
  upper = jnp.stack([ptr_ref[b + 1] for b in range(B)])[:, None]
  mask = ((rows >= lower) & (rows < upper)).astype(jnp.float32)
  partial = jax.lax.dot_general(
      mask, h_ref[...], (((1,), (0,)), ((), ())),
      preferred_element_type=jnp.float32)

  @pl.when(i == 0)
  def _():
    out_ref[...] = jnp.zeros_like(out_ref)

  out_ref[...] += partial


def _tc_rowsum(H, bag_ptr):
  return pl.pallas_call(
      _tc_rowsum_body,
      grid=(TC_ROWS // TC_BLK,),
      in_specs=[
          pl.BlockSpec(memory_space=pltpu.SMEM),
          pl.BlockSpec((TC_BLK, D), lambda i: (i + SC_ROWS // TC_BLK, 0)),
      ],
      out_specs=pl.BlockSpec((B, D), lambda i: (0, 0)),
      out_shape=jax.ShapeDtypeStruct((B, D), jnp.float32),
  )(bag_ptr, H)


def _combine_body(sc_ref, tc_ref, ptr_ref, out_ref):
  sums = jnp.sum(sc_ref[...], axis=0) + tc_ref[...]
  cnt = jnp.stack([ptr_ref[b + 1] - ptr_ref[b] for b in range(B)])
  denom = jnp.maximum(cnt.astype(jnp.float32), 1.0)[:, None]
  out_ref[...] = sums / denom


def kernel(H, bag_ptr):
  sc_partial = _partial_sums(H, bag_ptr)
  tc_partial = _tc_rowsum(H, bag_ptr)
  out = pl.pallas_call(
      _combine_body,
      out_shape=jax.ShapeDtypeStruct((B, D), jnp.float32),
      in_specs=[
          pl.BlockSpec(memory_space=pltpu.VMEM),
          pl.BlockSpec(memory_space=pltpu.VMEM),
          pl.BlockSpec(memory_space=pltpu.SMEM),
      ],
      out_specs=pl.BlockSpec(memory_space=pltpu.VMEM),
  )(sc_partial, tc_partial, bag_ptr)
  return out


# speedup vs baseline: 1.0513x; 1.0204x over previous
import jax
import jax.numpy as jnp
from jax import lax
from jax.experimental import pallas as pl
from jax.experimental.pallas import tpu as pltpu
from jax.experimental.pallas import tpu_sc as plsc
import functools

TOTAL = 32768
B = 16
D = 128
LANES = 16
NC = 1
NS = 16
NW = NC * NS

SC_ROWS = 4096
ROWS_PER_W = SC_ROWS // NW
CHUNK = 128
NCHUNK = ROWS_PER_W // CHUNK
DV = D // LANES

TC_ROWS = TOTAL - SC_ROWS
TC_BLK = 7168
SC_BASE = TC_ROWS


def _sc_partial_sums(h_hbm, ptr_hbm, out_hbm, ptr_v, buf0, buf1, acc,
                     shared, ptr_s, sem0, sem1):
  cid = lax.axis_index("c")
  sid = lax.axis_index("s")
  wid = sid * NC + cid
  base = SC_BASE + wid * ROWS_PER_W
  bufs = (buf0, buf1)
  sems = (sem0, sem1)

  pending = {}
  for c in range(min(2, NCHUNK)):
    pending[c] = pltpu.async_copy(
        h_hbm.at[pl.ds(base + c * CHUNK, CHUNK)], bufs[c % 2], sems[c % 2])

  pltpu.sync_copy(ptr_hbm.at[pl.ds(0, LANES)], ptr_v)
  ptr_vec = ptr_v[...]
  for b in range(B):
    ptr_s[b] = ptr_vec[b]
  ptr_s[B] = jnp.int32(TOTAL)

  zero = jnp.zeros((LANES,), jnp.float32)
  for b in range(B):
    for j in range(DV):
      acc[b, pl.ds(j * LANES, LANES)] = zero

  @pl.when(sid == 0)
  def _():
    pltpu.sync_copy(acc, shared)

  plsc.subcore_barrier()

  for c in range(NCHUNK):
    clo = base + c * CHUNK
    buf = bufs[c % 2]
    pending[c].wait()

    def bag_body(b, _):
      s_loc = jnp.clip(ptr_s[b] - clo, 0, CHUNK)
      e_loc = jnp.clip(ptr_s[b + 1] - clo, 0, CHUNK)

      @pl.when(e_loc > s_loc)
      def _():
        @plsc.parallel_loop(s_loc, e_loc, step=1, unroll=2,
                            carry=(zero,) * DV)
        def sums(r, carry):
          return tuple(carry[j] + buf[r, pl.ds(j * LANES, LANES)]
                       for j in range(DV))
        for j in range(DV):
          acc[b, pl.ds(j * LANES, LANES)] = (
              acc[b, pl.ds(j * LANES, LANES)] + sums[j])
      return 0

    lax.fori_loop(0, B, bag_body, 0)

    if c + 2 < NCHUNK:
      pending[c + 2] = pltpu.async_copy(
          h_hbm.at[pl.ds(base + (c + 2) * CHUNK, CHUNK)], buf, sems[c % 2])

  pltpu.sync_copy(acc, shared.at[lax.iota(jnp.int32, B)], add=True)
  plsc.subcore_barrier()

  @pl.when(sid == 0)
  def _():
    pltpu.sync_copy(shared, out_hbm.at[cid])


@functools.partial(
    pl.kernel,
    out_type=jax.ShapeDtypeStruct((NC, B, D), jnp.float32),
    mesh=plsc.VectorSubcoreMesh(core_axis_name="c", subcore_axis_name="s", num_cores=1),
    scratch_types=[
        pltpu.VMEM((LANES,), jnp.int32),
        pltpu.VMEM((CHUNK, D), jnp.float32),
        pltpu.VMEM((CHUNK, D), jnp.float32),
        pltpu.VMEM((B, D), jnp.float32),
        pltpu.VMEM_SHARED((B, D), jnp.float32),
        pltpu.SMEM((B + 1,), jnp.int32),
        pltpu.SemaphoreType.DMA,
        pltpu.SemaphoreType.DMA,
    ],
)
def _partial_sums(h_hbm, ptr_hbm, out_hbm, ptr_v, buf0, buf1, acc,
                  shared, ptr_s, sem0, sem1):
  _sc_partial_sums(h_hbm, ptr_hbm, out_hbm, ptr_v, buf0, buf1, acc,
                   shared, ptr_s, sem0, sem1)


def _tc_rowsum_body(ptr_ref, h_ref, out_ref):
  i = pl.program_id(0)
  rows = (i * TC_BLK
          + jax.lax.broadcasted_iota(jnp.int32, (1, TC_BLK), 1))
  lower = jnp.stack([ptr_ref[b] for b in range(B)])[:, None]
  upper = jnp.stack([ptr_ref[b + 1] for b in range(B)])[:, None]
  mask = ((rows >= lower) & (rows < upper)).astype(jnp.float32)
  partial = jax.lax.dot_general(
      mask, h_ref[...], (((1,), (0,)), ((), ())),
      preferred_element_type=jnp.float32)

  @pl.when(i == 0)
  def _():
    out_ref[...] = jnp.zeros_like(out_ref)

  out_ref[...] += partial


def _tc_rowsum(H, bag_ptr):
  return pl.pallas_call(
      _tc_rowsum_body,
      grid=(TC_ROWS // TC_BLK,),
      in_specs=[
          pl.BlockSpec(memory_space=pltpu.SMEM),
          pl.BlockSpec((TC_BLK, D), lambda i: (i, 0)),
      ],
      out_specs=pl.BlockSpec((B, D), lambda i: (0, 0)),
      out_shape=jax.ShapeDtypeStruct((B, D), jnp.float32),
  )(bag_ptr, H)


def _combine_body(sc_ref, tc_ref, ptr_ref, out_ref):
  sums = jnp.sum(sc_ref[...], axis=0) + tc_ref[...]
  cnt = jnp.stack([ptr_ref[b + 1] - ptr_ref[b] for b in range(B)])
  denom = jnp.maximum(cnt.astype(jnp.float32), 1.0)[:, None]
  out_ref[...] = sums / denom


def kernel(H, bag_ptr):
  sc_partial = _partial_sums(H, bag_ptr)
  tc_partial = _tc_rowsum(H, bag_ptr)
  out = pl.pallas_call(
      _combine_body,
      out_shape=jax.ShapeDtypeStruct((B, D), jnp.float32),
      in_specs=[
          pl.BlockSpec(memory_space=pltpu.VMEM),
          pl.BlockSpec(memory_space=pltpu.VMEM),
          pl.BlockSpec(memory_space=pltpu.SMEM),
      ],
      out_specs=pl.BlockSpec(memory_space=pltpu.VMEM),
  )(sc_partial, tc_partial, bag_ptr)
  return out


# speedup vs baseline: 1.0965x; 1.0429x over previous
import jax
import jax.numpy as jnp
from jax import lax
from jax.experimental import pallas as pl
from jax.experimental.pallas import tpu as pltpu
from jax.experimental.pallas import tpu_sc as plsc
import functools

TOTAL = 32768
B = 16
D = 128
LANES = 16
NC = 1
NS = 16
NW = NC * NS

SC_ROWS = 2048
ROWS_PER_W = SC_ROWS // NW
CHUNK = 64
NCHUNK = ROWS_PER_W // CHUNK
DV = D // LANES

TC_ROWS = TOTAL - SC_ROWS
TC_BLK = 7680
SC_BASE = TC_ROWS


def _sc_partial_sums(h_hbm, ptr_hbm, out_hbm, ptr_v, buf0, buf1, acc,
                     shared, ptr_s, sem0, sem1):
  cid = lax.axis_index("c")
  sid = lax.axis_index("s")
  wid = sid * NC + cid
  base = SC_BASE + wid * ROWS_PER_W
  bufs = (buf0, buf1)
  sems = (sem0, sem1)

  pending = {}
  for c in range(min(2, NCHUNK)):
    pending[c] = pltpu.async_copy(
        h_hbm.at[pl.ds(base + c * CHUNK, CHUNK)], bufs[c % 2], sems[c % 2])

  pltpu.sync_copy(ptr_hbm.at[pl.ds(0, LANES)], ptr_v)
  ptr_vec = ptr_v[...]
  for b in range(B):
    ptr_s[b] = ptr_vec[b]
  ptr_s[B] = jnp.int32(TOTAL)

  zero = jnp.zeros((LANES,), jnp.float32)
  for b in range(B):
    for j in range(DV):
      acc[b, pl.ds(j * LANES, LANES)] = zero

  @pl.when(sid == 0)
  def _():
    pltpu.sync_copy(acc, shared)

  plsc.subcore_barrier()

  for c in range(NCHUNK):
    clo = base + c * CHUNK
    buf = bufs[c % 2]
    pending[c].wait()

    def bag_body(b, _):
      s_loc = jnp.clip(ptr_s[b] - clo, 0, CHUNK)
      e_loc = jnp.clip(ptr_s[b + 1] - clo, 0, CHUNK)

      @pl.when(e_loc > s_loc)
      def _():
        @plsc.parallel_loop(s_loc, e_loc, step=1, unroll=2,
                            carry=(zero,) * DV)
        def sums(r, carry):
          return tuple(carry[j] + buf[r, pl.ds(j * LANES, LANES)]
                       for j in range(DV))
        for j in range(DV):
          acc[b, pl.ds(j * LANES, LANES)] = (
              acc[b, pl.ds(j * LANES, LANES)] + sums[j])
      return 0

    lax.fori_loop(0, B, bag_body, 0)

    if c + 2 < NCHUNK:
      pending[c + 2] = pltpu.async_copy(
          h_hbm.at[pl.ds(base + (c + 2) * CHUNK, CHUNK)], buf, sems[c % 2])

  pltpu.sync_copy(acc, shared.at[lax.iota(jnp.int32, B)], add=True)
  plsc.subcore_barrier()

  @pl.when(sid == 0)
  def _():
    pltpu.sync_copy(shared, out_hbm.at[cid])


@functools.partial(
    pl.kernel,
    out_type=jax.ShapeDtypeStruct((NC, B, D), jnp.float32),
    mesh=plsc.VectorSubcoreMesh(core_axis_name="c", subcore_axis_name="s", num_cores=1),
    scratch_types=[
        pltpu.VMEM((LANES,), jnp.int32),
        pltpu.VMEM((CHUNK, D), jnp.float32),
        pltpu.VMEM((CHUNK, D), jnp.float32),
        pltpu.VMEM((B, D), jnp.float32),
        pltpu.VMEM_SHARED((B, D), jnp.float32),
        pltpu.SMEM((B + 1,), jnp.int32),
        pltpu.SemaphoreType.DMA,
        pltpu.SemaphoreType.DMA,
    ],
)
def _partial_sums(h_hbm, ptr_hbm, out_hbm, ptr_v, buf0, buf1, acc,
                  shared, ptr_s, sem0, sem1):
  _sc_partial_sums(h_hbm, ptr_hbm, out_hbm, ptr_v, buf0, buf1, acc,
                   shared, ptr_s, sem0, sem1)


def _tc_rowsum_body(ptr_ref, h_ref, out_ref):
  i = pl.program_id(0)
  rows = (i * TC_BLK
          + jax.lax.broadcasted_iota(jnp.int32, (1, TC_BLK), 1))
  lower = jnp.stack([ptr_ref[b] for b in range(B)])[:, None]
  upper = jnp.stack([ptr_ref[b + 1] for b in range(B)])[:, None]
  mask = ((rows >= lower) & (rows < upper)).astype(jnp.float32)
  partial = jax.lax.dot_general(
      mask, h_ref[...], (((1,), (0,)), ((), ())),
      preferred_element_type=jnp.float32)

  @pl.when(i == 0)
  def _():
    out_ref[...] = jnp.zeros_like(out_ref)

  out_ref[...] += partial


def _tc_rowsum(H, bag_ptr):
  return pl.pallas_call(
      _tc_rowsum_body,
      grid=(TC_ROWS // TC_BLK,),
      in_specs=[
          pl.BlockSpec(memory_space=pltpu.SMEM),
          pl.BlockSpec((TC_BLK, D), lambda i: (i, 0)),
      ],
      out_specs=pl.BlockSpec((B, D), lambda i: (0, 0)),
      out_shape=jax.ShapeDtypeStruct((B, D), jnp.float32),
  )(bag_ptr, H)


def _combine_body(sc_ref, tc_ref, ptr_ref, out_ref):
  sums = jnp.sum(sc_ref[...], axis=0) + tc_ref[...]
  cnt = jnp.stack([ptr_ref[b + 1] - ptr_ref[b] for b in range(B)])
  denom = jnp.maximum(cnt.astype(jnp.float32), 1.0)[:, None]
  out_ref[...] = sums / denom


def kernel(H, bag_ptr):
  sc_partial = _partial_sums(H, bag_ptr)
  tc_partial = _tc_rowsum(H, bag_ptr)
  out = pl.pallas_call(
      _combine_body,
      out_shape=jax.ShapeDtypeStruct((B, D), jnp.float32),
      in_specs=[
          pl.BlockSpec(memory_space=pltpu.VMEM),
          pl.BlockSpec(memory_space=pltpu.VMEM),
          pl.BlockSpec(memory_space=pltpu.SMEM),
      ],
      out_specs=pl.BlockSpec(memory_space=pltpu.VMEM),
  )(sc_partial, tc_partial, bag_ptr)
  return out
